# Initial kernel scaffold; baseline (speedup 1.0000x reference)
import jax
import jax.numpy as jnp
from jax.experimental import pallas as pl


def kernel(asc_dec, cru_dec, des_dec, concat_index):
    raise NotImplementedError("write your pallas kernel here")


# speedup vs baseline: 1.1554x; 1.1554x over previous
import functools

import jax
import jax.numpy as jnp
from jax import lax
from jax.experimental import pallas as pl
from jax.experimental.pallas import tpu as pltpu
import jax.experimental.pallas.tpu_sc as plsc

N = 98304
D = 256
NC = 2
NS = 16
NW = NC * NS
ROWS_PER_W = N // NW
W = 128
N_WIN = ROWS_PER_W // W


def _gather_body(asc_hbm, cru_hbm, des_hbm, order_hbm, out_hbm,
                 idx_v, rows_v, sem):
    wid = lax.axis_index("s") * NC + lax.axis_index("c")
    base = wid * ROWS_PER_W

    def win(w, _):
        start = base + w * W
        pltpu.sync_copy(order_hbm.at[pl.ds(start, W)], idx_v)
        pltpu.async_copy(asc_hbm.at[idx_v], rows_v, sem).wait()
        pltpu.sync_copy(rows_v, out_hbm.at[pl.ds(start, W)])
        return 0

    lax.fori_loop(0, N_WIN, win, 0)


def kernel(asc_dec, cru_dec, des_dec, concat_index):
    order = jnp.argsort(concat_index)
    table = jnp.concatenate([asc_dec, cru_dec, des_dec], axis=0)

    mesh = plsc.VectorSubcoreMesh(core_axis_name="c", subcore_axis_name="s")
    gather = pl.kernel(
        _gather_body,
        mesh=mesh,
        out_type=jax.ShapeDtypeStruct((N, D), jnp.float32),
        scratch_types=[
            pltpu.VMEM((W,), jnp.int32),
            pltpu.VMEM((W, D), jnp.float32),
            pltpu.SemaphoreType.DMA,
        ],
    )
    return gather(table, cru_dec, des_dec, order)
